# baseline (speedup 1.0000x reference)
import jax
import jax.numpy as jnp
from jax.experimental import pallas as pl
from jax.experimental.pallas import tpu as pltpu

_HIDDEN = 128
_N_ACTION = 2


def _mlp_kernel_rows(x_ref, w1_ref, b1_ref, w2_ref, b2_ref, o_ref):
    h = jnp.dot(x_ref[...], w1_ref[...], preferred_element_type=jnp.float32)
    h = jnp.maximum(h + b1_ref[...], 0.0)
    y = jnp.dot(h, w2_ref[...], preferred_element_type=jnp.float32)
    o_ref[...] = jnp.tanh(y + b2_ref[...])


def _fallback_call(x, w1, b1, w2, b2, block_b):
    B, n_states = x.shape
    n_out = w2.shape[1]
    if B <= block_b:
        return pl.pallas_call(
            _mlp_kernel_rows,
            out_shape=jax.ShapeDtypeStruct((B, n_out), jnp.float32),
        )(x, w1, b1, w2, b2)
    nb = pl.cdiv(B, block_b)
    return pl.pallas_call(
        _mlp_kernel_rows,
        out_shape=jax.ShapeDtypeStruct((B, n_out), jnp.float32),
        grid=(nb,),
        in_specs=[
            pl.BlockSpec((block_b, n_states), lambda i: (i, 0)),
            pl.BlockSpec((n_states, w1.shape[1]), lambda i: (0, 0)),
            pl.BlockSpec((1, w1.shape[1]), lambda i: (0, 0)),
            pl.BlockSpec((w2.shape[0], n_out), lambda i: (0, 0)),
            pl.BlockSpec((1, n_out), lambda i: (0, 0)),
        ],
        out_specs=pl.BlockSpec((block_b, n_out), lambda i: (i, 0)),
        compiler_params=pltpu.CompilerParams(
            dimension_semantics=("parallel",)),
    )(x, w1, b1, w2, b2)


def _mlp_kernel_cols(xt_ref, w1t_ref, b1t_ref, w2t_ref, b2t_ref, o_ref):
    h = jnp.dot(w1t_ref[...], xt_ref[...], preferred_element_type=jnp.float32)
    h = jnp.maximum(h + b1t_ref[...], 0.0)
    y = jnp.dot(w2t_ref[...], h, preferred_element_type=jnp.float32)
    o_ref[...] = jnp.tanh(y + b2t_ref[...])


def kernel(x, w1, b1, w2p, b2p):
    B, n_states = x.shape
    w2 = w2p[:, :_N_ACTION]
    b2 = b2p[:, :_N_ACTION]

    block_c = 4096
    if B < block_c:
        return _fallback_call(x, w1, b1, w2, b2, 8192)

    xt = x.T.astype(jnp.bfloat16)
    w1t = w1.T.astype(jnp.bfloat16)
    b1t = b1.T
    w2t = w2.T
    b2t = b2.T

    nb = pl.cdiv(B, block_c)
    yt = pl.pallas_call(
        _mlp_kernel_cols,
        out_shape=jax.ShapeDtypeStruct((_N_ACTION, B), jnp.float32),
        grid=(nb,),
        in_specs=[
            pl.BlockSpec((n_states, block_c), lambda i: (0, i)),
            pl.BlockSpec((_HIDDEN, n_states), lambda i: (0, 0)),
            pl.BlockSpec((_HIDDEN, 1), lambda i: (0, 0)),
            pl.BlockSpec((_N_ACTION, _HIDDEN), lambda i: (0, 0)),
            pl.BlockSpec((_N_ACTION, 1), lambda i: (0, 0)),
        ],
        out_specs=pl.BlockSpec((_N_ACTION, block_c), lambda i: (0, i)),
        compiler_params=pltpu.CompilerParams(
            dimension_semantics=("parallel",)),
    )(xt, w1t, b1t, w2t, b2t)
    return yt.T


# speedup vs baseline: 1.6675x; 1.6675x over previous
import jax
import jax.numpy as jnp
from jax.experimental import pallas as pl
from jax.experimental.pallas import tpu as pltpu

_HIDDEN = 128
_N_ACTION = 2


def _mlp_kernel_rows(x_ref, w1_ref, b1_ref, w2_ref, b2_ref, o_ref):
    h = jnp.dot(x_ref[...], w1_ref[...], preferred_element_type=jnp.float32)
    h = jnp.maximum(h + b1_ref[...], 0.0)
    y = jnp.dot(h, w2_ref[...], preferred_element_type=jnp.float32)
    o_ref[...] = jnp.tanh(y + b2_ref[...])


def _fallback_call(x, w1, b1, w2, b2, block_b):
    B, n_states = x.shape
    n_out = w2.shape[1]
    if B <= block_b:
        return pl.pallas_call(
            _mlp_kernel_rows,
            out_shape=jax.ShapeDtypeStruct((B, n_out), jnp.float32),
        )(x, w1, b1, w2, b2)
    nb = pl.cdiv(B, block_b)
    return pl.pallas_call(
        _mlp_kernel_rows,
        out_shape=jax.ShapeDtypeStruct((B, n_out), jnp.float32),
        grid=(nb,),
        in_specs=[
            pl.BlockSpec((block_b, n_states), lambda i: (i, 0)),
            pl.BlockSpec((n_states, w1.shape[1]), lambda i: (0, 0)),
            pl.BlockSpec((1, w1.shape[1]), lambda i: (0, 0)),
            pl.BlockSpec((w2.shape[0], n_out), lambda i: (0, 0)),
            pl.BlockSpec((1, n_out), lambda i: (0, 0)),
        ],
        out_specs=pl.BlockSpec((block_b, n_out), lambda i: (i, 0)),
        compiler_params=pltpu.CompilerParams(
            dimension_semantics=("parallel",)),
    )(x, w1, b1, w2, b2)


_SUB_C = 4096


def _mlp_kernel_cols(xt_ref, w1t_ref, b1t_ref, w2t_ref, b2t_ref, o_ref):
    bc = xt_ref.shape[1]
    w1t = w1t_ref[...]
    b1t = b1t_ref[...]
    w2t = w2t_ref[...]
    b2t = b2t_ref[...]
    for j in range(bc // _SUB_C):
        xs = xt_ref[:, j * _SUB_C:(j + 1) * _SUB_C]
        h = jnp.dot(w1t, xs, preferred_element_type=jnp.float32)
        h = jnp.maximum(h + b1t, 0.0)
        y = jnp.dot(w2t, h, preferred_element_type=jnp.float32)
        o_ref[:, j * _SUB_C:(j + 1) * _SUB_C] = jnp.tanh(y + b2t)


def kernel(x, w1, b1, w2p, b2p):
    B, n_states = x.shape
    w2 = w2p[:, :_N_ACTION]
    b2 = b2p[:, :_N_ACTION]

    block_c = 16384
    if B < block_c:
        return _fallback_call(x, w1, b1, w2, b2, 8192)

    xt = x.T
    w1t = w1.T
    b1t = b1.T
    w2t = w2.T
    b2t = b2.T

    nb = pl.cdiv(B, block_c)
    yt = pl.pallas_call(
        _mlp_kernel_cols,
        out_shape=jax.ShapeDtypeStruct((_N_ACTION, B), jnp.float32),
        grid=(nb,),
        in_specs=[
            pl.BlockSpec((n_states, block_c), lambda i: (0, i)),
            pl.BlockSpec((_HIDDEN, n_states), lambda i: (0, 0)),
            pl.BlockSpec((_HIDDEN, 1), lambda i: (0, 0)),
            pl.BlockSpec((_N_ACTION, _HIDDEN), lambda i: (0, 0)),
            pl.BlockSpec((_N_ACTION, 1), lambda i: (0, 0)),
        ],
        out_specs=pl.BlockSpec((_N_ACTION, block_c), lambda i: (0, i)),
        compiler_params=pltpu.CompilerParams(
            dimension_semantics=("parallel",)),
    )(xt, w1t, b1t, w2t, b2t)
    return yt.T


# speedup vs baseline: 1.7845x; 1.0702x over previous
import jax
import jax.numpy as jnp
from jax.experimental import pallas as pl
from jax.experimental.pallas import tpu as pltpu

_HIDDEN = 128
_N_ACTION = 2


def _mlp_kernel_rows(x_ref, w1_ref, b1_ref, w2_ref, b2_ref, o_ref):
    h = jnp.dot(x_ref[...], w1_ref[...], preferred_element_type=jnp.float32)
    h = jnp.maximum(h + b1_ref[...], 0.0)
    y = jnp.dot(h, w2_ref[...], preferred_element_type=jnp.float32)
    o_ref[...] = jnp.tanh(y + b2_ref[...])


def _fallback_call(x, w1, b1, w2, b2, block_b):
    B, n_states = x.shape
    n_out = w2.shape[1]
    if B <= block_b:
        return pl.pallas_call(
            _mlp_kernel_rows,
            out_shape=jax.ShapeDtypeStruct((B, n_out), jnp.float32),
        )(x, w1, b1, w2, b2)
    nb = pl.cdiv(B, block_b)
    return pl.pallas_call(
        _mlp_kernel_rows,
        out_shape=jax.ShapeDtypeStruct((B, n_out), jnp.float32),
        grid=(nb,),
        in_specs=[
            pl.BlockSpec((block_b, n_states), lambda i: (i, 0)),
            pl.BlockSpec((n_states, w1.shape[1]), lambda i: (0, 0)),
            pl.BlockSpec((1, w1.shape[1]), lambda i: (0, 0)),
            pl.BlockSpec((w2.shape[0], n_out), lambda i: (0, 0)),
            pl.BlockSpec((1, n_out), lambda i: (0, 0)),
        ],
        out_specs=pl.BlockSpec((block_b, n_out), lambda i: (i, 0)),
        compiler_params=pltpu.CompilerParams(
            dimension_semantics=("parallel",)),
    )(x, w1, b1, w2, b2)


_SUB_C = 4096


def _mlp_kernel_cols(xt_ref, w1t_ref, b1t_ref, w2t_ref, b2t_ref, o_ref):
    bc = xt_ref.shape[1]
    w1t = w1t_ref[...]
    b1t = b1t_ref[...]
    w2t = w2t_ref[...]
    b2t = b2t_ref[...]
    for j in range(bc // _SUB_C):
        xs = xt_ref[:, j * _SUB_C:(j + 1) * _SUB_C]
        h = jnp.dot(w1t, xs, preferred_element_type=jnp.float32)
        h = jnp.maximum(h + b1t, 0.0)
        y = jnp.dot(w2t, h, preferred_element_type=jnp.float32)
        o_ref[:, j * _SUB_C:(j + 1) * _SUB_C] = jnp.tanh(y + b2t)


def kernel(x, w1, b1, w2p, b2p):
    B, n_states = x.shape
    w2 = w2p[:, :_N_ACTION]
    b2 = b2p[:, :_N_ACTION]

    block_c = 65536
    if B < block_c:
        return _fallback_call(x, w1, b1, w2, b2, 8192)

    xt = x.T
    w1t = w1.T
    b1t = b1.T
    w2t = w2.T
    b2t = b2.T

    nb = pl.cdiv(B, block_c)
    yt = pl.pallas_call(
        _mlp_kernel_cols,
        out_shape=jax.ShapeDtypeStruct((_N_ACTION, B), jnp.float32),
        grid=(nb,),
        in_specs=[
            pl.BlockSpec((n_states, block_c), lambda i: (0, i)),
            pl.BlockSpec((_HIDDEN, n_states), lambda i: (0, 0)),
            pl.BlockSpec((_HIDDEN, 1), lambda i: (0, 0)),
            pl.BlockSpec((_N_ACTION, _HIDDEN), lambda i: (0, 0)),
            pl.BlockSpec((_N_ACTION, 1), lambda i: (0, 0)),
        ],
        out_specs=pl.BlockSpec((_N_ACTION, block_c), lambda i: (0, i)),
        compiler_params=pltpu.CompilerParams(
            dimension_semantics=("parallel",)),
    )(xt, w1t, b1t, w2t, b2t)
    return yt.T


# speedup vs baseline: 1.8006x; 1.0090x over previous
import jax
import jax.numpy as jnp
from jax.experimental import pallas as pl
from jax.experimental.pallas import tpu as pltpu

_HIDDEN = 128
_N_ACTION = 2


def _mlp_kernel_rows(x_ref, w1_ref, b1_ref, w2_ref, b2_ref, o_ref):
    h = jnp.dot(x_ref[...], w1_ref[...], preferred_element_type=jnp.float32)
    h = jnp.maximum(h + b1_ref[...], 0.0)
    y = jnp.dot(h, w2_ref[...], preferred_element_type=jnp.float32)
    o_ref[...] = jnp.tanh(y + b2_ref[...])


def _fallback_call(x, w1, b1, w2, b2, block_b):
    B, n_states = x.shape
    n_out = w2.shape[1]
    if B <= block_b:
        return pl.pallas_call(
            _mlp_kernel_rows,
            out_shape=jax.ShapeDtypeStruct((B, n_out), jnp.float32),
        )(x, w1, b1, w2, b2)
    nb = pl.cdiv(B, block_b)
    return pl.pallas_call(
        _mlp_kernel_rows,
        out_shape=jax.ShapeDtypeStruct((B, n_out), jnp.float32),
        grid=(nb,),
        in_specs=[
            pl.BlockSpec((block_b, n_states), lambda i: (i, 0)),
            pl.BlockSpec((n_states, w1.shape[1]), lambda i: (0, 0)),
            pl.BlockSpec((1, w1.shape[1]), lambda i: (0, 0)),
            pl.BlockSpec((w2.shape[0], n_out), lambda i: (0, 0)),
            pl.BlockSpec((1, n_out), lambda i: (0, 0)),
        ],
        out_specs=pl.BlockSpec((block_b, n_out), lambda i: (i, 0)),
        compiler_params=pltpu.CompilerParams(
            dimension_semantics=("parallel",)),
    )(x, w1, b1, w2, b2)


_SUB_C = 8192


def _mlp_kernel_cols(xt_ref, w1t_ref, b1t_ref, w2t_ref, b2t_ref, o_ref):
    bc = xt_ref.shape[1]
    w1t = w1t_ref[...]
    b1t = b1t_ref[...]
    w2t = w2t_ref[...]
    b2t = b2t_ref[...]
    for j in range(bc // _SUB_C):
        xs = xt_ref[:, j * _SUB_C:(j + 1) * _SUB_C]
        h = jnp.dot(w1t, xs, preferred_element_type=jnp.float32)
        h = jnp.maximum(h + b1t, 0.0)
        y = jnp.dot(w2t, h, preferred_element_type=jnp.float32)
        o_ref[:, j * _SUB_C:(j + 1) * _SUB_C] = jnp.tanh(y + b2t)


def kernel(x, w1, b1, w2p, b2p):
    B, n_states = x.shape
    w2 = w2p[:, :_N_ACTION]
    b2 = b2p[:, :_N_ACTION]

    block_c = 65536
    if B < block_c:
        return _fallback_call(x, w1, b1, w2, b2, 8192)

    xt = x.T
    w1t = w1.T
    b1t = b1.T
    w2t = w2.T
    b2t = b2.T

    nb = pl.cdiv(B, block_c)
    yt = pl.pallas_call(
        _mlp_kernel_cols,
        out_shape=jax.ShapeDtypeStruct((_N_ACTION, B), jnp.float32),
        grid=(nb,),
        in_specs=[
            pl.BlockSpec((n_states, block_c), lambda i: (0, i)),
            pl.BlockSpec((_HIDDEN, n_states), lambda i: (0, 0)),
            pl.BlockSpec((_HIDDEN, 1), lambda i: (0, 0)),
            pl.BlockSpec((_N_ACTION, _HIDDEN), lambda i: (0, 0)),
            pl.BlockSpec((_N_ACTION, 1), lambda i: (0, 0)),
        ],
        out_specs=pl.BlockSpec((_N_ACTION, block_c), lambda i: (0, i)),
        compiler_params=pltpu.CompilerParams(
            dimension_semantics=("parallel",)),
    )(xt, w1t, b1t, w2t, b2t)
    return yt.T


# speedup vs baseline: 1.8066x; 1.0033x over previous
import jax
import jax.numpy as jnp
from jax.experimental import pallas as pl
from jax.experimental.pallas import tpu as pltpu

_HIDDEN = 128
_N_ACTION = 2


def _mlp_kernel_rows(x_ref, w1_ref, b1_ref, w2_ref, b2_ref, o_ref):
    h = jnp.dot(x_ref[...], w1_ref[...], preferred_element_type=jnp.float32)
    h = jnp.maximum(h + b1_ref[...], 0.0)
    y = jnp.dot(h, w2_ref[...], preferred_element_type=jnp.float32)
    o_ref[...] = jnp.tanh(y + b2_ref[...])


def _fallback_call(x, w1, b1, w2, b2, block_b):
    B, n_states = x.shape
    n_out = w2.shape[1]
    if B <= block_b:
        return pl.pallas_call(
            _mlp_kernel_rows,
            out_shape=jax.ShapeDtypeStruct((B, n_out), jnp.float32),
        )(x, w1, b1, w2, b2)
    nb = pl.cdiv(B, block_b)
    return pl.pallas_call(
        _mlp_kernel_rows,
        out_shape=jax.ShapeDtypeStruct((B, n_out), jnp.float32),
        grid=(nb,),
        in_specs=[
            pl.BlockSpec((block_b, n_states), lambda i: (i, 0)),
            pl.BlockSpec((n_states, w1.shape[1]), lambda i: (0, 0)),
            pl.BlockSpec((1, w1.shape[1]), lambda i: (0, 0)),
            pl.BlockSpec((w2.shape[0], n_out), lambda i: (0, 0)),
            pl.BlockSpec((1, n_out), lambda i: (0, 0)),
        ],
        out_specs=pl.BlockSpec((block_b, n_out), lambda i: (i, 0)),
        compiler_params=pltpu.CompilerParams(
            dimension_semantics=("parallel",)),
    )(x, w1, b1, w2, b2)


_SUB_C = 8192


def _mlp_kernel_cols(xt_ref, w1t_ref, b1t_ref, w2t_ref, b2t_ref, o_ref):
    bc = xt_ref.shape[1]
    w1t = w1t_ref[...]
    b1t = b1t_ref[...]
    w2t = w2t_ref[...]
    b2t = b2t_ref[...]
    for j in range(bc // _SUB_C):
        xs = xt_ref[:, j * _SUB_C:(j + 1) * _SUB_C]
        h = jnp.dot(w1t, xs, preferred_element_type=jnp.float32)
        h = jnp.maximum(h + b1t, 0.0)
        y = jnp.dot(w2t, h, preferred_element_type=jnp.float32)
        o_ref[:, j * _SUB_C:(j + 1) * _SUB_C] = jnp.tanh(y + b2t)


def kernel(x, w1, b1, w2p, b2p):
    B, n_states = x.shape
    w2 = w2p[:, :_N_ACTION]
    b2 = b2p[:, :_N_ACTION]

    block_c = 131072
    if B < block_c:
        return _fallback_call(x, w1, b1, w2, b2, 8192)

    xt = x.T
    w1t = w1.T
    b1t = b1.T
    w2t = w2.T
    b2t = b2.T

    nb = pl.cdiv(B, block_c)
    yt = pl.pallas_call(
        _mlp_kernel_cols,
        out_shape=jax.ShapeDtypeStruct((_N_ACTION, B), jnp.float32),
        grid=(nb,),
        in_specs=[
            pl.BlockSpec((n_states, block_c), lambda i: (0, i)),
            pl.BlockSpec((_HIDDEN, n_states), lambda i: (0, 0)),
            pl.BlockSpec((_HIDDEN, 1), lambda i: (0, 0)),
            pl.BlockSpec((_N_ACTION, _HIDDEN), lambda i: (0, 0)),
            pl.BlockSpec((_N_ACTION, 1), lambda i: (0, 0)),
        ],
        out_specs=pl.BlockSpec((_N_ACTION, block_c), lambda i: (0, i)),
        compiler_params=pltpu.CompilerParams(
            dimension_semantics=("parallel",)),
    )(xt, w1t, b1t, w2t, b2t)
    return yt.T


# speedup vs baseline: 1.8076x; 1.0005x over previous
import jax
import jax.numpy as jnp
from jax.experimental import pallas as pl
from jax.experimental.pallas import tpu as pltpu

_HIDDEN = 128
_N_ACTION = 2


def _mlp_kernel_rows(x_ref, w1_ref, b1_ref, w2_ref, b2_ref, o_ref):
    h = jnp.dot(x_ref[...], w1_ref[...], preferred_element_type=jnp.float32)
    h = jnp.maximum(h + b1_ref[...], 0.0)
    y = jnp.dot(h, w2_ref[...], preferred_element_type=jnp.float32)
    o_ref[...] = jnp.tanh(y + b2_ref[...])


def _fallback_call(x, w1, b1, w2, b2, block_b):
    B, n_states = x.shape
    n_out = w2.shape[1]
    if B <= block_b:
        return pl.pallas_call(
            _mlp_kernel_rows,
            out_shape=jax.ShapeDtypeStruct((B, n_out), jnp.float32),
        )(x, w1, b1, w2, b2)
    nb = pl.cdiv(B, block_b)
    return pl.pallas_call(
        _mlp_kernel_rows,
        out_shape=jax.ShapeDtypeStruct((B, n_out), jnp.float32),
        grid=(nb,),
        in_specs=[
            pl.BlockSpec((block_b, n_states), lambda i: (i, 0)),
            pl.BlockSpec((n_states, w1.shape[1]), lambda i: (0, 0)),
            pl.BlockSpec((1, w1.shape[1]), lambda i: (0, 0)),
            pl.BlockSpec((w2.shape[0], n_out), lambda i: (0, 0)),
            pl.BlockSpec((1, n_out), lambda i: (0, 0)),
        ],
        out_specs=pl.BlockSpec((block_b, n_out), lambda i: (i, 0)),
        compiler_params=pltpu.CompilerParams(
            dimension_semantics=("parallel",)),
    )(x, w1, b1, w2, b2)


_SUB_C = 8192


def _mlp_kernel_cols(xt_ref, w1t_ref, b1t_ref, w2t_ref, b2t_ref, o_ref):
    bc = xt_ref.shape[1]
    w1t = w1t_ref[...]
    b1t = b1t_ref[...]
    w2t = w2t_ref[...]
    b2t = b2t_ref[...]
    for j in range(bc // _SUB_C):
        xs = xt_ref[:, j * _SUB_C:(j + 1) * _SUB_C]
        h = jnp.dot(w1t, xs, preferred_element_type=jnp.float32)
        h = jnp.maximum(h + b1t, 0.0)
        y = jnp.dot(w2t, h, preferred_element_type=jnp.float32)
        o_ref[:, j * _SUB_C:(j + 1) * _SUB_C] = jnp.tanh(y + b2t)


def kernel(x, w1, b1, w2p, b2p):
    B, n_states = x.shape
    w2 = w2p[:, :_N_ACTION]
    b2 = b2p[:, :_N_ACTION]

    block_c = 131072
    if B < block_c:
        return _fallback_call(x, w1, b1, w2, b2, 8192)

    xt = x.T
    w1t = w1.T
    b1t = b1.T
    w2t = w2.T
    b2t = b2.T

    nb = pl.cdiv(B, block_c)
    yt = pl.pallas_call(
        _mlp_kernel_cols,
        out_shape=jax.ShapeDtypeStruct((_N_ACTION, B), jnp.float32),
        grid=(nb,),
        in_specs=[
            pl.BlockSpec((n_states, block_c), lambda i: (0, i)),
            pl.BlockSpec((_HIDDEN, n_states), lambda i: (0, 0)),
            pl.BlockSpec((_HIDDEN, 1), lambda i: (0, 0)),
            pl.BlockSpec((_N_ACTION, _HIDDEN), lambda i: (0, 0)),
            pl.BlockSpec((_N_ACTION, 1), lambda i: (0, 0)),
        ],
        out_specs=pl.BlockSpec((_N_ACTION, block_c), lambda i: (0, i)),
        compiler_params=pltpu.CompilerParams(
            dimension_semantics=("arbitrary",)),
    )(xt, w1t, b1t, w2t, b2t)
    return yt.T


# speedup vs baseline: 1.8091x; 1.0008x over previous
import jax
import jax.numpy as jnp
from jax.experimental import pallas as pl
from jax.experimental.pallas import tpu as pltpu

_HIDDEN = 128
_N_ACTION = 2


def _mlp_kernel_rows(x_ref, w1_ref, b1_ref, w2_ref, b2_ref, o_ref):
    h = jnp.dot(x_ref[...], w1_ref[...], preferred_element_type=jnp.float32)
    h = jnp.maximum(h + b1_ref[...], 0.0)
    y = jnp.dot(h, w2_ref[...], preferred_element_type=jnp.float32)
    o_ref[...] = jnp.tanh(y + b2_ref[...])


def _fallback_call(x, w1, b1, w2, b2, block_b):
    B, n_states = x.shape
    n_out = w2.shape[1]
    if B <= block_b:
        return pl.pallas_call(
            _mlp_kernel_rows,
            out_shape=jax.ShapeDtypeStruct((B, n_out), jnp.float32),
        )(x, w1, b1, w2, b2)
    nb = pl.cdiv(B, block_b)
    return pl.pallas_call(
        _mlp_kernel_rows,
        out_shape=jax.ShapeDtypeStruct((B, n_out), jnp.float32),
        grid=(nb,),
        in_specs=[
            pl.BlockSpec((block_b, n_states), lambda i: (i, 0)),
            pl.BlockSpec((n_states, w1.shape[1]), lambda i: (0, 0)),
            pl.BlockSpec((1, w1.shape[1]), lambda i: (0, 0)),
            pl.BlockSpec((w2.shape[0], n_out), lambda i: (0, 0)),
            pl.BlockSpec((1, n_out), lambda i: (0, 0)),
        ],
        out_specs=pl.BlockSpec((block_b, n_out), lambda i: (i, 0)),
        compiler_params=pltpu.CompilerParams(
            dimension_semantics=("parallel",)),
    )(x, w1, b1, w2, b2)


_SUB_C = 8192


def _mlp_kernel_cols(xt_ref, w1t_ref, b1t_ref, w2t_ref, b2t_ref, o_ref):
    bc = xt_ref.shape[1]
    w1t = w1t_ref[...]
    b1t = b1t_ref[...]
    w2t = w2t_ref[...]
    b2t = b2t_ref[...]
    for j in range(bc // _SUB_C):
        xs = xt_ref[:, j * _SUB_C:(j + 1) * _SUB_C]
        h = jnp.dot(w1t, xs, preferred_element_type=jnp.float32)
        h = jnp.maximum(h + b1t, 0.0)
        y = jnp.dot(w2t, h, preferred_element_type=jnp.float32)
        o_ref[:, j * _SUB_C:(j + 1) * _SUB_C] = jnp.tanh(y + b2t)


def kernel(x, w1, b1, w2p, b2p):
    B, n_states = x.shape
    w2 = w2p[:, :_N_ACTION]
    b2 = b2p[:, :_N_ACTION]

    block_c = 131072
    if B < block_c:
        return _fallback_call(x, w1, b1, w2, b2, 8192)

    xt = x.T
    w1t = w1.T
    b1t = b1.T
    w2t = w2.T
    b2t = b2.T

    nb = pl.cdiv(B, block_c)
    yt = pl.pallas_call(
        _mlp_kernel_cols,
        out_shape=jax.ShapeDtypeStruct((_N_ACTION, B), jnp.float32),
        grid=(nb,),
        in_specs=[
            pl.BlockSpec((n_states, block_c), lambda i: (0, i)),
            pl.BlockSpec((_HIDDEN, n_states), lambda i: (0, 0)),
            pl.BlockSpec((_HIDDEN, 1), lambda i: (0, 0)),
            pl.BlockSpec((_N_ACTION, _HIDDEN), lambda i: (0, 0)),
            pl.BlockSpec((_N_ACTION, 1), lambda i: (0, 0)),
        ],
        out_specs=pl.BlockSpec((_N_ACTION, block_c), lambda i: (0, i)),
        compiler_params=pltpu.CompilerParams(
            dimension_semantics=("parallel",)),
    )(xt, w1t, b1t, w2t, b2t)
    return yt.T
